# baseline (speedup 1.0000x reference)
import functools
import jax
import jax.numpy as jnp
from jax import lax
from jax.experimental import pallas as pl
from jax.experimental.pallas import tpu as pltpu
from jax.experimental.pallas import tpu_sc as plsc

NN = 10000
EE = 320000
DD = 128
HH = 256
NP = 10240
NB = NP // 128
NC = 2
NS = 16
NT = NC * NS
CH = 128
EPT = ((EE + NT - 1) // NT + CH - 1) // CH * CH
EP = EPT * NT
NCHUNK = EPT // CH
RPT = NP // NS
NCH0 = 88
NCH1 = 2 * NCHUNK - NCH0
NCHMAX = max(NCH0, NCH1)
CACC = 1152

_f32 = jnp.float32
_i32 = jnp.int32

@functools.cache
def _sc_mesh():
    return plsc.VectorSubcoreMesh(core_axis_name="c", subcore_axis_name="s",
                                  num_cores=NC, num_subcores=NS)


def _tile_ids():
    return lax.axis_index("c"), lax.axis_index("s")


def _fill_f32(ref, n, value):
    def body(i, _):
        ref[pl.ds(i * 16, 16)] = jnp.full((16,), value, _f32)
        return 0
    lax.fori_loop(0, n // 16, body, 0)


def _fill_i32(ref, n, value):
    def body(i, _):
        ref[pl.ds(i * 16, 16)] = jnp.full((16,), value, _i32)
        return 0
    lax.fori_loop(0, n // 16, body, 0)


def _k1_body(src_hbm, dst_hbm, out_hbm, ones_v, sidx_v, didx_v, zero_v,
             semo, semi, acc_o, acc_i):
    c, s = _tile_ids()
    tid = c * NS + s
    _fill_f32(zero_v, RPT, 0.0)
    pltpu.sync_copy(zero_v, acc_o.at[pl.ds(s * RPT, RPT)])
    pltpu.sync_copy(zero_v, acc_i.at[pl.ds(s * RPT, RPT)])
    _fill_f32(ones_v, CH, 1.0)
    pltpu.sync_copy(src_hbm.at[tid], sidx_v)
    pltpu.sync_copy(dst_hbm.at[tid], didx_v)
    plsc.subcore_barrier()

    def chunk(k, _):
        pltpu.async_copy(ones_v, acc_o.at[sidx_v.at[k]], semo, add=True)
        pltpu.async_copy(ones_v, acc_i.at[didx_v.at[k]], semi, add=True)
        return 0

    lax.fori_loop(0, NCHUNK, chunk, 0)

    def drain(k, _):
        pltpu.make_async_copy(ones_v, acc_o.at[sidx_v.at[k]], semo).wait()
        pltpu.make_async_copy(ones_v, acc_i.at[didx_v.at[k]], semi).wait()
        return 0

    lax.fori_loop(0, NCHUNK, drain, 0)
    plsc.subcore_barrier()
    pltpu.sync_copy(acc_o.at[pl.ds(s * RPT, RPT)],
                    out_hbm.at[c, 0, pl.ds(s * RPT, RPT)])
    pltpu.sync_copy(acc_i.at[pl.ds(s * RPT, RPT)],
                    out_hbm.at[c, 1, pl.ds(s * RPT, RPT)])


@functools.cache
def _k1():
  return pl.kernel(
    _k1_body,
    out_type=jax.ShapeDtypeStruct((NC, 2, NP), _f32),
    mesh=_sc_mesh(),
    compiler_params=pltpu.CompilerParams(needs_layout_passes=False),
    scratch_types=[
        pltpu.VMEM((CH,), _f32),
        pltpu.VMEM((NCHUNK, CH), _i32),
        pltpu.VMEM((NCHUNK, CH), _i32),
        pltpu.VMEM((RPT,), _f32),
        pltpu.SemaphoreType.DMA,
        pltpu.SemaphoreType.DMA,
        pltpu.VMEM_SHARED((NP,), _f32),
        pltpu.VMEM_SHARED((NP,), _f32),
    ],
)


def _k3_body(src_hbm, dst_hbm, xs_hbm, out_hbm, sidx_v, didx_v, rows_v,
             semg, semd, semsc, agg_s):
    c, s = _tile_ids()
    tid = c * NS + s
    ncht = jnp.where(c == 0, NCH0, NCH1)
    pltpu.sync_copy(src_hbm.at[tid], sidx_v)

    def zfill(i, _):
        for j in range(8):
            rows_v[0, i, pl.ds(j * 16, 16)] = jnp.zeros((16,), _f32)
        return 0

    lax.fori_loop(0, 128, zfill, 0)

    def zcopy(r, _):
        pltpu.sync_copy(rows_v.at[0], agg_s.at[pl.ds(s * RPT + r * 128, 128)])
        return 0

    lax.fori_loop(0, RPT // 128, zcopy, 0)
    plsc.subcore_barrier()

    pltpu.async_copy(dst_hbm.at[tid, 0], didx_v.at[0], semd)
    pltpu.async_copy(xs_hbm.at[sidx_v.at[0]], rows_v.at[0], semg)

    def chunk(k, _):
        @pl.when(k < ncht)
        def _():
            @pl.when(k >= 2)
            def _():
                pltpu.make_async_copy(
                    rows_v.at[k % 2], agg_s.at[didx_v.at[k % 2]],
                    semsc).wait()
            pltpu.async_copy(dst_hbm.at[tid, k], didx_v.at[k % 2], semd)
            pltpu.async_copy(xs_hbm.at[sidx_v.at[k]], rows_v.at[k % 2], semg)
        pltpu.make_async_copy(xs_hbm.at[sidx_v.at[k - 1]],
                              rows_v.at[(k - 1) % 2], semg).wait()
        pltpu.make_async_copy(dst_hbm.at[tid, k - 1],
                              didx_v.at[(k - 1) % 2], semd).wait()
        pltpu.async_copy(rows_v.at[(k - 1) % 2],
                         agg_s.at[didx_v.at[(k - 1) % 2]], semsc, add=True)
        return 0

    lax.fori_loop(1, ncht + 1, chunk, 0)
    pltpu.make_async_copy(rows_v.at[(ncht - 2) % 2],
                          agg_s.at[didx_v.at[(ncht - 2) % 2]], semsc).wait()
    pltpu.make_async_copy(rows_v.at[(ncht - 1) % 2],
                          agg_s.at[didx_v.at[(ncht - 1) % 2]], semsc).wait()
    plsc.subcore_barrier()
    pltpu.sync_copy(agg_s.at[pl.ds(s * RPT, RPT)],
                    out_hbm.at[c, pl.ds(s * RPT, RPT)])


@functools.cache
def _k3():
  return pl.kernel(
    _k3_body,
    out_type=jax.ShapeDtypeStruct((NC, NP, DD), _f32),
    mesh=_sc_mesh(),
    compiler_params=pltpu.CompilerParams(needs_layout_passes=False),
    scratch_types=[
        pltpu.VMEM((NCHMAX, CH), _i32),
        pltpu.VMEM((2, CH), _i32),
        pltpu.VMEM((2, CH, DD), _f32),
        pltpu.SemaphoreType.DMA,
        pltpu.SemaphoreType.DMA,
        pltpu.SemaphoreType.DMA,
        pltpu.VMEM_SHARED((NP, DD), _f32),
    ],
)


def _k5_body(src_hbm, dst_hbm, s_hbm, out_hbm, s_tab, sidx_v, didx_v,
             vals_v, zero_v, sem, acc):
    c, s = _tile_ids()
    tid = c * NS + s
    _fill_f32(zero_v, RPT, 0.0)
    pltpu.sync_copy(zero_v, acc.at[pl.ds(s * RPT, RPT)])
    pltpu.sync_copy(s_hbm, s_tab)
    pltpu.sync_copy(src_hbm.at[tid], sidx_v)
    pltpu.sync_copy(dst_hbm.at[tid], didx_v)

    def gath(i, _):
        k = i // (CH // 16)
        j = i % (CH // 16)
        sv = sidx_v[k, pl.ds(j * 16, 16)]
        vals_v[k, pl.ds(j * 16, 16)] = plsc.load_gather(s_tab, [sv])
        return 0

    lax.fori_loop(0, EPT // 16, gath, 0)
    plsc.subcore_barrier()

    def chunk(k, _):
        pltpu.async_copy(vals_v.at[k], acc.at[didx_v.at[k]], sem, add=True)
        return 0

    lax.fori_loop(0, NCHUNK, chunk, 0)

    def drain(k, _):
        pltpu.make_async_copy(vals_v.at[k], acc.at[didx_v.at[k]], sem).wait()
        return 0

    lax.fori_loop(0, NCHUNK, drain, 0)
    plsc.subcore_barrier()
    pltpu.sync_copy(acc.at[pl.ds(s * RPT, RPT)],
                    out_hbm.at[c, pl.ds(s * RPT, RPT)])


@functools.cache
def _k5():
  return pl.kernel(
    _k5_body,
    out_type=jax.ShapeDtypeStruct((NC, NP), _f32),
    mesh=_sc_mesh(),
    compiler_params=pltpu.CompilerParams(needs_layout_passes=False),
    scratch_types=[
        pltpu.VMEM((NP,), _f32),
        pltpu.VMEM((NCHUNK, CH), _i32),
        pltpu.VMEM((NCHUNK, CH), _i32),
        pltpu.VMEM((NCHUNK, CH), _f32),
        pltpu.VMEM((RPT,), _f32),
        pltpu.SemaphoreType.DMA,
        pltpu.VMEM_SHARED((NP,), _f32),
    ],
)


def _k7_body(idx32_hbm, src_hbm, dst_hbm, agg0_hbm, agg1_hbm, nd_hbm, ns_hbm,
             c_out, agg32_out, nd32_out, ns32_out,
             map_tab, val_tab, idx32_v, idxs_v, idxd_v, cidx_v, ones_v,
             zero_v, rows32_v, vec32_v, sem, sem2, cacc):
    c, s = _tile_ids()
    _fill_f32(zero_v, CACC, 0.0)
    _fill_f32(ones_v, CH, 1.0)

    @pl.when(s == 0)
    def _():
        pltpu.sync_copy(zero_v, cacc)

    _fill_i32(map_tab, NP, 32)
    pltpu.sync_copy(idx32_hbm, idx32_v)
    for j in range(2):
        iv = idx32_v[pl.ds(j * 16, 16)]
        plsc.store_scatter(map_tab, [iv],
                           lax.iota(_i32, 16) + jnp.int32(j * 16))
    tid = c * NS + s
    pltpu.sync_copy(src_hbm.at[tid], idxs_v)
    pltpu.sync_copy(dst_hbm.at[tid], idxd_v)

    def cchunk(k, nfired):
        def gath(j, cnt):
            s16 = idxs_v[k, pl.ds(j * 16, 16)]
            d16 = idxd_v[k, pl.ds(j * 16, 16)]
            ms = plsc.load_gather(map_tab, [s16])
            md = plsc.load_gather(map_tab, [d16])
            valid = (ms < 32) & (md < 32)
            dump = jnp.int32(1024 + j * 16) + lax.iota(_i32, 16)
            flat = jnp.where(valid, ms * 32 + md, dump)
            cidx_v[k, pl.ds(j * 16, 16)] = flat
            return cnt + jnp.sum(valid.astype(_i32))

        nval = lax.fori_loop(0, CH // 16, gath, jnp.int32(0))

        @pl.when(nval > 0)
        def _():
            pltpu.async_copy(ones_v, cacc.at[cidx_v.at[k]], sem2, add=True)

        return nfired + jnp.where(nval > 0, 1, 0)

    nfired = lax.fori_loop(0, NCHUNK, cchunk, jnp.int32(0))

    def drain(k, _):
        @pl.when(k < nfired)
        def _():
            pltpu.make_async_copy(ones_v, cacc.at[cidx_v.at[0]], sem2).wait()
        return 0

    lax.fori_loop(0, NCHUNK, drain, 0)
    plsc.subcore_barrier()

    @pl.when(s == 0)
    def _():
        pltpu.sync_copy(cacc.at[pl.ds(0, 1024)], c_out.at[c])

    @pl.when((c == 0) & (s == 4))
    def _():
        pltpu.async_copy(agg0_hbm.at[idx32_v], rows32_v, sem).wait()
        pltpu.sync_copy(rows32_v, agg32_out.at[0])

    @pl.when((c == 0) & (s == 5))
    def _():
        pltpu.async_copy(agg1_hbm.at[idx32_v], rows32_v, sem).wait()
        pltpu.sync_copy(rows32_v, agg32_out.at[1])

    @pl.when((c == 0) & (s == 6))
    def _():
        pltpu.sync_copy(nd_hbm, val_tab)
        for j in range(2):
            iv = idx32_v[pl.ds(j * 16, 16)]
            vec32_v[pl.ds(j * 16, 16)] = plsc.load_gather(val_tab, [iv])
        pltpu.sync_copy(vec32_v, nd32_out)

    @pl.when((c == 0) & (s == 7))
    def _():
        pltpu.sync_copy(ns_hbm, val_tab)
        for j in range(2):
            iv = idx32_v[pl.ds(j * 16, 16)]
            vec32_v[pl.ds(j * 16, 16)] = plsc.load_gather(val_tab, [iv])
        pltpu.sync_copy(vec32_v, ns32_out)


@functools.cache
def _k7():
  return pl.kernel(
    _k7_body,
    out_type=(
        jax.ShapeDtypeStruct((NC, 1024), _f32),
        jax.ShapeDtypeStruct((2, 32, DD), _f32),
        jax.ShapeDtypeStruct((32,), _f32),
        jax.ShapeDtypeStruct((32,), _f32),
    ),
    mesh=_sc_mesh(),
    compiler_params=pltpu.CompilerParams(needs_layout_passes=False),
    scratch_types=[
        pltpu.VMEM((NP,), _i32),
        pltpu.VMEM((NP,), _f32),
        pltpu.VMEM((32,), _i32),
        pltpu.VMEM((NCHUNK, CH), _i32),
        pltpu.VMEM((NCHUNK, CH), _i32),
        pltpu.VMEM((NCHUNK, CH), _i32),
        pltpu.VMEM((CH,), _f32),
        pltpu.VMEM((CACC,), _f32),
        pltpu.VMEM((32, DD), _f32),
        pltpu.VMEM((32,), _f32),
        pltpu.SemaphoreType.DMA,
        pltpu.SemaphoreType.DMA,
        pltpu.VMEM_SHARED((CACC,), _f32),
    ],
)


def _leaky(z):
    return jnp.where(z >= 0, z, 0.01 * z)


def _k2_body(degh_ref, x_ref, xs_ref, ns_ref, nd_ref):
    dh = degh_ref[...]
    dego = dh[0, 0] + dh[1, 0]
    degi = dh[0, 1] + dh[1, 1]
    ns = lax.rsqrt(jnp.maximum(dego, 1.0))
    nd = lax.rsqrt(jnp.maximum(degi, 1.0))
    xs_ref[...] = x_ref[...] * ns
    ns_ref[...] = ns
    nd_ref[...] = nd


_k2 = pl.pallas_call(
    _k2_body,
    grid=(NB,),
    in_specs=[
        pl.BlockSpec((NC, 2, 128, 1), lambda i: (0, 0, i, 0)),
        pl.BlockSpec((128, DD), lambda i: (i, 0)),
    ],
    out_specs=[
        pl.BlockSpec((128, DD), lambda i: (i, 0)),
        pl.BlockSpec((128, 1), lambda i: (i, 0)),
        pl.BlockSpec((128, 1), lambda i: (i, 0)),
    ],
    out_shape=[
        jax.ShapeDtypeStruct((NP, DD), _f32),
        jax.ShapeDtypeStruct((NP, 1), _f32),
        jax.ShapeDtypeStruct((NP, 1), _f32),
    ],
)


def _k4_body(agg_ref, nd_ref, ns_ref, w1_ref, b1_ref, s1_ref, s_ref):
    a = agg_ref[0] + agg_ref[1]
    z = jnp.dot(a * nd_ref[...], w1_ref[...],
                preferred_element_type=_f32) + b1_ref[...]
    h = _leaky(z)
    s_ref[...] = ns_ref[...] * jnp.dot(h, s1_ref[...],
                                       preferred_element_type=_f32)


_k4 = pl.pallas_call(
    _k4_body,
    grid=(NB,),
    in_specs=[
        pl.BlockSpec((NC, 128, DD), lambda i: (0, i, 0)),
        pl.BlockSpec((128, 1), lambda i: (i, 0)),
        pl.BlockSpec((128, 1), lambda i: (i, 0)),
        pl.BlockSpec((DD, HH), lambda i: (0, 0)),
        pl.BlockSpec((1, HH), lambda i: (0, 0)),
        pl.BlockSpec((HH, 1), lambda i: (0, 0)),
    ],
    out_specs=pl.BlockSpec((128, 1), lambda i: (i, 0)),
    out_shape=jax.ShapeDtypeStruct((NP, 1), _f32),
)


def _k6_body(shist_ref, nd_ref, sb1_ref, topi_ref, topv_ref):
    sh = shist_ref[...]
    sc = (sh[0] + sh[1]) * nd_ref[...] + sb1_ref[0, 0]
    flat = (lax.broadcasted_iota(_i32, (NB, 128), 0) * 128
            + lax.broadcasted_iota(_i32, (NB, 128), 1))
    sc = jnp.where(flat < NN, sc, -jnp.inf)
    lane = lax.broadcasted_iota(_i32, (1, 128), 1)

    def step(j, carry):
        scv, iv, vv = carry
        m = jnp.max(scv)
        am = jnp.min(jnp.where(scv == m, flat, jnp.int32(2 ** 30)))
        iv = jnp.where(lane == j, am, iv)
        vv = jnp.where(lane == j, m, vv)
        scv = jnp.where(flat == am, -jnp.inf, scv)
        return scv, iv, vv

    _, iv, vv = lax.fori_loop(
        0, 32, step,
        (sc, jnp.zeros((1, 128), _i32), jnp.zeros((1, 128), _f32)))
    topi_ref[...] = iv
    topv_ref[...] = vv


_k6 = pl.pallas_call(
    _k6_body,
    out_shape=[
        jax.ShapeDtypeStruct((1, 128), _i32),
        jax.ShapeDtypeStruct((1, 128), _f32),
    ],
)


def _topk_small(score_col, n, k):
    rid = lax.broadcasted_iota(_i32, (n, 1), 0)
    rowk = lax.broadcasted_iota(_i32, (k, n), 0)
    colk = lax.broadcasted_iota(_i32, (k, n), 1)
    rowk1 = lax.broadcasted_iota(_i32, (k, 1), 0)
    P = jnp.zeros((k, n), _f32)
    tv = jnp.zeros((k, 1), _f32)
    sc = score_col
    for j in range(k):
        m = jnp.max(sc)
        am = jnp.min(jnp.where(sc == m, rid, jnp.int32(2 ** 30)))
        P = jnp.where((rowk == j) & (colk == am), 1.0, P)
        tv = jnp.where(rowk1 == j, m, tv)
        sc = jnp.where(rid == am, -jnp.inf, sc)
    return P, tv


def _colsum(Cm, n):
    ones = jnp.ones((n, 1), _f32)
    return lax.dot_general(Cm, ones, (((0,), (0,)), ((), ())),
                           preferred_element_type=_f32)


def _k8_body(cp_ref, agg32_ref, nd32_ref, ns32_ref, topv32_ref,
             w1_ref, b1_ref, w2_ref, b2_ref, w3_ref, b3_ref,
             s2_ref, sb2_ref, s3_ref, sb3_ref,
             wd1_ref, bd1_ref, wd2_ref, bd2_ref, out_ref):
    C = cp_ref[0] + cp_ref[1]
    agg32 = agg32_ref[0] + agg32_ref[1]
    nd32 = nd32_ref[...]
    ns32 = ns32_ref[...]

    h1 = _leaky(jnp.dot(agg32 * nd32, w1_ref[...],
                        preferred_element_type=_f32) + b1_ref[...])
    x1 = h1 * jnp.tanh(topv32_ref[...])
    r1 = jnp.concatenate(
        [jnp.sum(x1, axis=0, keepdims=True),
         jnp.max(x1, axis=0, keepdims=True)], axis=1)

    def level(xk, Cm, n, k, W, b, S, sb):
        dego = jnp.sum(Cm, axis=1, keepdims=True)
        degi = _colsum(Cm, n)
        ns_ = lax.rsqrt(jnp.maximum(dego, 1.0))
        nd_ = lax.rsqrt(jnp.maximum(degi, 1.0))
        agg = lax.dot_general(Cm, xk * ns_, (((0,), (0,)), ((), ())),
                              preferred_element_type=_f32) * nd_
        h = _leaky(jnp.dot(agg, W, preferred_element_type=_f32) + b)
        s_node = ns_ * jnp.dot(h, S, preferred_element_type=_f32)
        score = nd_ * lax.dot_general(Cm, s_node, (((0,), (0,)), ((), ())),
                                      preferred_element_type=_f32) + sb
        P, tv = _topk_small(score, n, k)
        xnext = jnp.dot(P, h, preferred_element_type=_f32) * jnp.tanh(tv)
        t = jnp.dot(P, Cm, preferred_element_type=_f32)
        Cnext = lax.dot_general(t, P, (((1,), (1,)), ((), ())),
                                preferred_element_type=_f32)
        r = jnp.concatenate(
            [jnp.sum(xnext, axis=0, keepdims=True),
             jnp.max(xnext, axis=0, keepdims=True)], axis=1)
        return xnext, Cnext, r

    x2, C3, r2 = level(x1, C, 32, 16, w2_ref[...], b2_ref[...],
                       s2_ref[...], sb2_ref[0, 0])
    _, _, r3 = level(x2, C3, 16, 8, w3_ref[...], b3_ref[...],
                     s3_ref[...], sb3_ref[0, 0])

    merged = jnp.concatenate([r1, r2, r3], axis=1)
    d1 = _leaky(jnp.dot(merged, wd1_ref[...],
                        preferred_element_type=_f32) + bd1_ref[...])
    logits = jnp.dot(d1, wd2_ref[...], preferred_element_type=_f32) \
        + bd2_ref[...]
    out_ref[...] = 1.0 / (1.0 + jnp.exp(-logits))


_k8 = pl.pallas_call(
    _k8_body,
    out_shape=jax.ShapeDtypeStruct((1, 2), _f32),
)


def kernel(x, edge_index, W1, b1, W2, b2, W3, b3, S1, sb1, S2, sb2, S3, sb3,
           Wd1, bd1, Wd2, bd2):
    src = edge_index[0].astype(_i32)
    dst = edge_index[1].astype(_i32)
    pad = jnp.full((EP - EE,), NN, _i32)
    src_p = jnp.concatenate([src, pad])
    dst_p = jnp.concatenate([dst, pad])
    x_pad = jnp.concatenate([x, jnp.zeros((NP - NN, DD), _f32)], axis=0)

    src_t = src_p.reshape(NT, NCHUNK, CH)
    dst_t = dst_p.reshape(NT, NCHUNK, CH)
    degh = _k1()(src_t, dst_t)
    xs, ns_col, nd_col = _k2(degh.reshape(NC, 2, NP, 1), x_pad)
    def split3(a):
        p0 = a[:16 * NCH0 * CH].reshape(16, NCH0, CH)
        p0 = jnp.concatenate(
            [p0, jnp.full((16, NCHMAX - NCH0, CH), NN, _i32)], axis=1)
        p1 = a[16 * NCH0 * CH:].reshape(16, NCH1, CH)
        p1 = jnp.concatenate(
            [p1, jnp.full((16, NCHMAX - NCH1, CH), NN, _i32)], axis=1)
        return jnp.concatenate([p0, p1], axis=0)

    aggp = _k3()(split3(src_p), split3(dst_p), xs)
    s_col = _k4(aggp, nd_col, ns_col, W1, b1.reshape(1, HH),
                S1)
    shist = _k5()(src_t, dst_t, s_col.reshape(NP))
    topi, topv = _k6(shist.reshape(NC, NB, 128),
                     nd_col.reshape(NB, 128),
                     sb1.reshape(1, 1))
    idx32 = topi[0, :32]
    cp, agg32p, nd32, ns32 = _k7()(idx32, src_t, dst_t, aggp[0], aggp[1],
                                   nd_col.reshape(NP), ns_col.reshape(NP))
    out = _k8(cp.reshape(NC, 32, 32), agg32p, nd32.reshape(32, 1),
              ns32.reshape(32, 1), topv[0, :32].reshape(32, 1),
              W1, b1.reshape(1, HH), W2, b2.reshape(1, HH),
              W3, b3.reshape(1, HH), S2, sb2.reshape(1, 1),
              S3, sb3.reshape(1, 1), Wd1, bd1.reshape(1, 128),
              Wd2, bd2.reshape(1, 2))
    return out


# speedup vs baseline: 1.0770x; 1.0770x over previous
import functools
import jax
import jax.numpy as jnp
from jax import lax
from jax.experimental import pallas as pl
from jax.experimental.pallas import tpu as pltpu
from jax.experimental.pallas import tpu_sc as plsc

NN = 10000
EE = 320000
DD = 128
HH = 256
NP = 10240
NB = NP // 128
NC = 2
NS = 16
NT = NC * NS
CH = 128
EPT = ((EE + NT - 1) // NT + CH - 1) // CH * CH
EP = EPT * NT
NCHUNK = EPT // CH
RPT = NP // NS
NCH0 = 96
NCH1 = 2 * NCHUNK - NCH0
NCHMAX = max(NCH0, NCH1)
CACC = 1152

_f32 = jnp.float32
_i32 = jnp.int32

@functools.cache
def _sc_mesh():
    return plsc.VectorSubcoreMesh(core_axis_name="c", subcore_axis_name="s",
                                  num_cores=NC, num_subcores=NS)


def _tile_ids():
    return lax.axis_index("c"), lax.axis_index("s")


def _fill_f32(ref, n, value):
    def body(i, _):
        ref[pl.ds(i * 16, 16)] = jnp.full((16,), value, _f32)
        return 0
    lax.fori_loop(0, n // 16, body, 0)


def _fill_i32(ref, n, value):
    def body(i, _):
        ref[pl.ds(i * 16, 16)] = jnp.full((16,), value, _i32)
        return 0
    lax.fori_loop(0, n // 16, body, 0)


def _k1_body(src_hbm, dst_hbm, out_hbm, ones_v, sidx_v, didx_v, zero_v,
             semo, semi, acc_o, acc_i):
    c, s = _tile_ids()
    tid = c * NS + s
    _fill_f32(zero_v, RPT, 0.0)
    pltpu.sync_copy(zero_v, acc_o.at[pl.ds(s * RPT, RPT)])
    pltpu.sync_copy(zero_v, acc_i.at[pl.ds(s * RPT, RPT)])
    _fill_f32(ones_v, CH, 1.0)
    pltpu.sync_copy(src_hbm.at[tid], sidx_v)
    pltpu.sync_copy(dst_hbm.at[tid], didx_v)
    plsc.subcore_barrier()

    def chunk(k, _):
        pltpu.async_copy(ones_v, acc_o.at[sidx_v.at[k]], semo, add=True)
        pltpu.async_copy(ones_v, acc_i.at[didx_v.at[k]], semi, add=True)
        return 0

    lax.fori_loop(0, NCHUNK, chunk, 0)

    def drain(k, _):
        pltpu.make_async_copy(ones_v, acc_o.at[sidx_v.at[k]], semo).wait()
        pltpu.make_async_copy(ones_v, acc_i.at[didx_v.at[k]], semi).wait()
        return 0

    lax.fori_loop(0, NCHUNK, drain, 0)
    plsc.subcore_barrier()
    pltpu.sync_copy(acc_o.at[pl.ds(s * RPT, RPT)],
                    out_hbm.at[c, 0, pl.ds(s * RPT, RPT)])
    pltpu.sync_copy(acc_i.at[pl.ds(s * RPT, RPT)],
                    out_hbm.at[c, 1, pl.ds(s * RPT, RPT)])


@functools.cache
def _k1():
  return pl.kernel(
    _k1_body,
    out_type=jax.ShapeDtypeStruct((NC, 2, NP), _f32),
    mesh=_sc_mesh(),
    compiler_params=pltpu.CompilerParams(needs_layout_passes=False),
    scratch_types=[
        pltpu.VMEM((CH,), _f32),
        pltpu.VMEM((NCHUNK, CH), _i32),
        pltpu.VMEM((NCHUNK, CH), _i32),
        pltpu.VMEM((RPT,), _f32),
        pltpu.SemaphoreType.DMA,
        pltpu.SemaphoreType.DMA,
        pltpu.VMEM_SHARED((NP,), _f32),
        pltpu.VMEM_SHARED((NP,), _f32),
    ],
)


def _k3_body(src_hbm, dst_hbm, xs_hbm, out_hbm, sidx_v, didx_v, rows_v,
             semg, semd, semsc, agg_s):
    c, s = _tile_ids()
    tid = c * NS + s
    ncht = jnp.where(c == 0, NCH0, NCH1)
    pltpu.sync_copy(src_hbm.at[tid], sidx_v)

    def zfill(i, _):
        for j in range(8):
            rows_v[0, i, pl.ds(j * 16, 16)] = jnp.zeros((16,), _f32)
        return 0

    lax.fori_loop(0, 128, zfill, 0)

    def zcopy(r, _):
        pltpu.sync_copy(rows_v.at[0], agg_s.at[pl.ds(s * RPT + r * 128, 128)])
        return 0

    lax.fori_loop(0, RPT // 128, zcopy, 0)
    plsc.subcore_barrier()

    pltpu.async_copy(dst_hbm.at[tid, 0], didx_v.at[0], semd)
    pltpu.async_copy(xs_hbm.at[sidx_v.at[0]], rows_v.at[0], semg)

    def chunk(k, _):
        @pl.when(k < ncht)
        def _():
            @pl.when(k >= 2)
            def _():
                pltpu.make_async_copy(
                    rows_v.at[k % 2], agg_s.at[didx_v.at[k % 2]],
                    semsc).wait()
            pltpu.async_copy(dst_hbm.at[tid, k], didx_v.at[k % 2], semd)
            pltpu.async_copy(xs_hbm.at[sidx_v.at[k]], rows_v.at[k % 2], semg)
        pltpu.make_async_copy(xs_hbm.at[sidx_v.at[k - 1]],
                              rows_v.at[(k - 1) % 2], semg).wait()
        pltpu.make_async_copy(dst_hbm.at[tid, k - 1],
                              didx_v.at[(k - 1) % 2], semd).wait()
        pltpu.async_copy(rows_v.at[(k - 1) % 2],
                         agg_s.at[didx_v.at[(k - 1) % 2]], semsc, add=True)
        return 0

    lax.fori_loop(1, ncht + 1, chunk, 0)
    pltpu.make_async_copy(rows_v.at[(ncht - 2) % 2],
                          agg_s.at[didx_v.at[(ncht - 2) % 2]], semsc).wait()
    pltpu.make_async_copy(rows_v.at[(ncht - 1) % 2],
                          agg_s.at[didx_v.at[(ncht - 1) % 2]], semsc).wait()
    plsc.subcore_barrier()
    pltpu.sync_copy(agg_s.at[pl.ds(s * RPT, RPT)],
                    out_hbm.at[c, pl.ds(s * RPT, RPT)])


@functools.cache
def _k3():
  return pl.kernel(
    _k3_body,
    out_type=jax.ShapeDtypeStruct((NC, NP, DD), _f32),
    mesh=_sc_mesh(),
    compiler_params=pltpu.CompilerParams(needs_layout_passes=False),
    scratch_types=[
        pltpu.VMEM((NCHMAX, CH), _i32),
        pltpu.VMEM((2, CH), _i32),
        pltpu.VMEM((2, CH, DD), _f32),
        pltpu.SemaphoreType.DMA,
        pltpu.SemaphoreType.DMA,
        pltpu.SemaphoreType.DMA,
        pltpu.VMEM_SHARED((NP, DD), _f32),
    ],
)


def _k5_body(src_hbm, dst_hbm, s_hbm, out_hbm, s_tab, sidx_v, didx_v,
             vals_v, zero_v, sem, acc):
    c, s = _tile_ids()
    tid = c * NS + s
    _fill_f32(zero_v, RPT, 0.0)
    pltpu.sync_copy(zero_v, acc.at[pl.ds(s * RPT, RPT)])
    pltpu.sync_copy(s_hbm, s_tab)
    pltpu.sync_copy(src_hbm.at[tid], sidx_v)
    pltpu.sync_copy(dst_hbm.at[tid], didx_v)

    def gath(i, _):
        k = i // (CH // 16)
        j = i % (CH // 16)
        sv = sidx_v[k, pl.ds(j * 16, 16)]
        vals_v[k, pl.ds(j * 16, 16)] = plsc.load_gather(s_tab, [sv])
        return 0

    lax.fori_loop(0, EPT // 16, gath, 0)
    plsc.subcore_barrier()

    def chunk(k, _):
        pltpu.async_copy(vals_v.at[k], acc.at[didx_v.at[k]], sem, add=True)
        return 0

    lax.fori_loop(0, NCHUNK, chunk, 0)

    def drain(k, _):
        pltpu.make_async_copy(vals_v.at[k], acc.at[didx_v.at[k]], sem).wait()
        return 0

    lax.fori_loop(0, NCHUNK, drain, 0)
    plsc.subcore_barrier()
    pltpu.sync_copy(acc.at[pl.ds(s * RPT, RPT)],
                    out_hbm.at[c, pl.ds(s * RPT, RPT)])


@functools.cache
def _k5():
  return pl.kernel(
    _k5_body,
    out_type=jax.ShapeDtypeStruct((NC, NP), _f32),
    mesh=_sc_mesh(),
    compiler_params=pltpu.CompilerParams(needs_layout_passes=False),
    scratch_types=[
        pltpu.VMEM((NP,), _f32),
        pltpu.VMEM((NCHUNK, CH), _i32),
        pltpu.VMEM((NCHUNK, CH), _i32),
        pltpu.VMEM((NCHUNK, CH), _f32),
        pltpu.VMEM((RPT,), _f32),
        pltpu.SemaphoreType.DMA,
        pltpu.VMEM_SHARED((NP,), _f32),
    ],
)


def _k7_body(idx32_hbm, src_hbm, dst_hbm, agg0_hbm, agg1_hbm, nd_hbm, ns_hbm,
             c_out, agg32_out, nd32_out, ns32_out,
             map_tab, val_tab, idx32_v, idxs_v, idxd_v, cidx_v, ones_v,
             zero_v, rows32_v, vec32_v, sem, sem2, cacc):
    c, s = _tile_ids()
    _fill_f32(zero_v, CACC, 0.0)
    _fill_f32(ones_v, CH, 1.0)

    @pl.when(s == 0)
    def _():
        pltpu.sync_copy(zero_v, cacc)

    _fill_i32(map_tab, NP, 32)
    pltpu.sync_copy(idx32_hbm, idx32_v)
    for j in range(2):
        iv = idx32_v[pl.ds(j * 16, 16)]
        plsc.store_scatter(map_tab, [iv],
                           lax.iota(_i32, 16) + jnp.int32(j * 16))
    tid = c * NS + s
    pltpu.sync_copy(src_hbm.at[tid], idxs_v)
    pltpu.sync_copy(dst_hbm.at[tid], idxd_v)

    def cchunk(k, nfired):
        def gath(j, cnt):
            s16 = idxs_v[k, pl.ds(j * 16, 16)]
            d16 = idxd_v[k, pl.ds(j * 16, 16)]
            ms = plsc.load_gather(map_tab, [s16])
            md = plsc.load_gather(map_tab, [d16])
            valid = (ms < 32) & (md < 32)
            dump = jnp.int32(1024 + j * 16) + lax.iota(_i32, 16)
            flat = jnp.where(valid, ms * 32 + md, dump)
            cidx_v[k, pl.ds(j * 16, 16)] = flat
            return cnt + jnp.sum(valid.astype(_i32))

        nval = lax.fori_loop(0, CH // 16, gath, jnp.int32(0))

        @pl.when(nval > 0)
        def _():
            pltpu.async_copy(ones_v, cacc.at[cidx_v.at[k]], sem2, add=True)

        return nfired + jnp.where(nval > 0, 1, 0)

    nfired = lax.fori_loop(0, NCHUNK, cchunk, jnp.int32(0))

    def drain(k, _):
        @pl.when(k < nfired)
        def _():
            pltpu.make_async_copy(ones_v, cacc.at[cidx_v.at[0]], sem2).wait()
        return 0

    lax.fori_loop(0, NCHUNK, drain, 0)
    plsc.subcore_barrier()

    @pl.when(s == 0)
    def _():
        pltpu.sync_copy(cacc.at[pl.ds(0, 1024)], c_out.at[c])

    @pl.when((c == 0) & (s == 4))
    def _():
        pltpu.async_copy(agg0_hbm.at[idx32_v], rows32_v, sem).wait()
        pltpu.sync_copy(rows32_v, agg32_out.at[0])

    @pl.when((c == 0) & (s == 5))
    def _():
        pltpu.async_copy(agg1_hbm.at[idx32_v], rows32_v, sem).wait()
        pltpu.sync_copy(rows32_v, agg32_out.at[1])

    @pl.when((c == 0) & (s == 6))
    def _():
        pltpu.sync_copy(nd_hbm, val_tab)
        for j in range(2):
            iv = idx32_v[pl.ds(j * 16, 16)]
            vec32_v[pl.ds(j * 16, 16)] = plsc.load_gather(val_tab, [iv])
        pltpu.sync_copy(vec32_v, nd32_out)

    @pl.when((c == 0) & (s == 7))
    def _():
        pltpu.sync_copy(ns_hbm, val_tab)
        for j in range(2):
            iv = idx32_v[pl.ds(j * 16, 16)]
            vec32_v[pl.ds(j * 16, 16)] = plsc.load_gather(val_tab, [iv])
        pltpu.sync_copy(vec32_v, ns32_out)


@functools.cache
def _k7():
  return pl.kernel(
    _k7_body,
    out_type=(
        jax.ShapeDtypeStruct((NC, 1024), _f32),
        jax.ShapeDtypeStruct((2, 32, DD), _f32),
        jax.ShapeDtypeStruct((32,), _f32),
        jax.ShapeDtypeStruct((32,), _f32),
    ),
    mesh=_sc_mesh(),
    compiler_params=pltpu.CompilerParams(needs_layout_passes=False),
    scratch_types=[
        pltpu.VMEM((NP,), _i32),
        pltpu.VMEM((NP,), _f32),
        pltpu.VMEM((32,), _i32),
        pltpu.VMEM((NCHUNK, CH), _i32),
        pltpu.VMEM((NCHUNK, CH), _i32),
        pltpu.VMEM((NCHUNK, CH), _i32),
        pltpu.VMEM((CH,), _f32),
        pltpu.VMEM((CACC,), _f32),
        pltpu.VMEM((32, DD), _f32),
        pltpu.VMEM((32,), _f32),
        pltpu.SemaphoreType.DMA,
        pltpu.SemaphoreType.DMA,
        pltpu.VMEM_SHARED((CACC,), _f32),
    ],
)


def _leaky(z):
    return jnp.where(z >= 0, z, 0.01 * z)


def _k2_body(degh_ref, x_ref, xs_ref, ns_ref, nd_ref):
    dh = degh_ref[...]
    dego = dh[0, 0] + dh[1, 0]
    degi = dh[0, 1] + dh[1, 1]
    ns = lax.rsqrt(jnp.maximum(dego, 1.0))
    nd = lax.rsqrt(jnp.maximum(degi, 1.0))
    xs_ref[...] = x_ref[...] * ns
    ns_ref[...] = ns
    nd_ref[...] = nd


_k2 = pl.pallas_call(
    _k2_body,
    grid=(NB,),
    in_specs=[
        pl.BlockSpec((NC, 2, 128, 1), lambda i: (0, 0, i, 0)),
        pl.BlockSpec((128, DD), lambda i: (i, 0)),
    ],
    out_specs=[
        pl.BlockSpec((128, DD), lambda i: (i, 0)),
        pl.BlockSpec((128, 1), lambda i: (i, 0)),
        pl.BlockSpec((128, 1), lambda i: (i, 0)),
    ],
    out_shape=[
        jax.ShapeDtypeStruct((NP, DD), _f32),
        jax.ShapeDtypeStruct((NP, 1), _f32),
        jax.ShapeDtypeStruct((NP, 1), _f32),
    ],
)


def _k4_body(agg_ref, nd_ref, ns_ref, w1_ref, b1_ref, s1_ref, s_ref):
    a = agg_ref[0] + agg_ref[1]
    z = jnp.dot(a * nd_ref[...], w1_ref[...],
                preferred_element_type=_f32) + b1_ref[...]
    h = _leaky(z)
    s_ref[...] = ns_ref[...] * jnp.dot(h, s1_ref[...],
                                       preferred_element_type=_f32)


_k4 = pl.pallas_call(
    _k4_body,
    grid=(NB,),
    in_specs=[
        pl.BlockSpec((NC, 128, DD), lambda i: (0, i, 0)),
        pl.BlockSpec((128, 1), lambda i: (i, 0)),
        pl.BlockSpec((128, 1), lambda i: (i, 0)),
        pl.BlockSpec((DD, HH), lambda i: (0, 0)),
        pl.BlockSpec((1, HH), lambda i: (0, 0)),
        pl.BlockSpec((HH, 1), lambda i: (0, 0)),
    ],
    out_specs=pl.BlockSpec((128, 1), lambda i: (i, 0)),
    out_shape=jax.ShapeDtypeStruct((NP, 1), _f32),
)


def _k6_body(shist_ref, nd_ref, sb1_ref, topi_ref, topv_ref):
    sh = shist_ref[...]
    sc = (sh[0] + sh[1]) * nd_ref[...] + sb1_ref[0, 0]
    flat = (lax.broadcasted_iota(_i32, (NB, 128), 0) * 128
            + lax.broadcasted_iota(_i32, (NB, 128), 1))
    sc = jnp.where(flat < NN, sc, -jnp.inf)
    lane = lax.broadcasted_iota(_i32, (1, 128), 1)

    def step(j, carry):
        scv, iv, vv = carry
        m = jnp.max(scv)
        am = jnp.min(jnp.where(scv == m, flat, jnp.int32(2 ** 30)))
        iv = jnp.where(lane == j, am, iv)
        vv = jnp.where(lane == j, m, vv)
        scv = jnp.where(flat == am, -jnp.inf, scv)
        return scv, iv, vv

    _, iv, vv = lax.fori_loop(
        0, 32, step,
        (sc, jnp.zeros((1, 128), _i32), jnp.zeros((1, 128), _f32)))
    topi_ref[...] = iv
    topv_ref[...] = vv


_k6 = pl.pallas_call(
    _k6_body,
    out_shape=[
        jax.ShapeDtypeStruct((1, 128), _i32),
        jax.ShapeDtypeStruct((1, 128), _f32),
    ],
)


def _topk_small(score_col, n, k):
    rid = lax.broadcasted_iota(_i32, (n, 1), 0)
    rowk = lax.broadcasted_iota(_i32, (k, n), 0)
    colk = lax.broadcasted_iota(_i32, (k, n), 1)
    rowk1 = lax.broadcasted_iota(_i32, (k, 1), 0)
    P = jnp.zeros((k, n), _f32)
    tv = jnp.zeros((k, 1), _f32)
    sc = score_col
    for j in range(k):
        m = jnp.max(sc)
        am = jnp.min(jnp.where(sc == m, rid, jnp.int32(2 ** 30)))
        P = jnp.where((rowk == j) & (colk == am), 1.0, P)
        tv = jnp.where(rowk1 == j, m, tv)
        sc = jnp.where(rid == am, -jnp.inf, sc)
    return P, tv


def _colsum(Cm, n):
    ones = jnp.ones((n, 1), _f32)
    return lax.dot_general(Cm, ones, (((0,), (0,)), ((), ())),
                           preferred_element_type=_f32)


def _k8_body(cp_ref, agg32_ref, nd32_ref, ns32_ref, topv32_ref,
             w1_ref, b1_ref, w2_ref, b2_ref, w3_ref, b3_ref,
             s2_ref, sb2_ref, s3_ref, sb3_ref,
             wd1_ref, bd1_ref, wd2_ref, bd2_ref, out_ref):
    C = cp_ref[0] + cp_ref[1]
    agg32 = agg32_ref[0] + agg32_ref[1]
    nd32 = nd32_ref[...]
    ns32 = ns32_ref[...]

    h1 = _leaky(jnp.dot(agg32 * nd32, w1_ref[...],
                        preferred_element_type=_f32) + b1_ref[...])
    x1 = h1 * jnp.tanh(topv32_ref[...])
    r1 = jnp.concatenate(
        [jnp.sum(x1, axis=0, keepdims=True),
         jnp.max(x1, axis=0, keepdims=True)], axis=1)

    def level(xk, Cm, n, k, W, b, S, sb):
        dego = jnp.sum(Cm, axis=1, keepdims=True)
        degi = _colsum(Cm, n)
        ns_ = lax.rsqrt(jnp.maximum(dego, 1.0))
        nd_ = lax.rsqrt(jnp.maximum(degi, 1.0))
        agg = lax.dot_general(Cm, xk * ns_, (((0,), (0,)), ((), ())),
                              preferred_element_type=_f32) * nd_
        h = _leaky(jnp.dot(agg, W, preferred_element_type=_f32) + b)
        s_node = ns_ * jnp.dot(h, S, preferred_element_type=_f32)
        score = nd_ * lax.dot_general(Cm, s_node, (((0,), (0,)), ((), ())),
                                      preferred_element_type=_f32) + sb
        P, tv = _topk_small(score, n, k)
        xnext = jnp.dot(P, h, preferred_element_type=_f32) * jnp.tanh(tv)
        t = jnp.dot(P, Cm, preferred_element_type=_f32)
        Cnext = lax.dot_general(t, P, (((1,), (1,)), ((), ())),
                                preferred_element_type=_f32)
        r = jnp.concatenate(
            [jnp.sum(xnext, axis=0, keepdims=True),
             jnp.max(xnext, axis=0, keepdims=True)], axis=1)
        return xnext, Cnext, r

    x2, C3, r2 = level(x1, C, 32, 16, w2_ref[...], b2_ref[...],
                       s2_ref[...], sb2_ref[0, 0])
    _, _, r3 = level(x2, C3, 16, 8, w3_ref[...], b3_ref[...],
                     s3_ref[...], sb3_ref[0, 0])

    merged = jnp.concatenate([r1, r2, r3], axis=1)
    d1 = _leaky(jnp.dot(merged, wd1_ref[...],
                        preferred_element_type=_f32) + bd1_ref[...])
    logits = jnp.dot(d1, wd2_ref[...], preferred_element_type=_f32) \
        + bd2_ref[...]
    out_ref[...] = 1.0 / (1.0 + jnp.exp(-logits))


_k8 = pl.pallas_call(
    _k8_body,
    out_shape=jax.ShapeDtypeStruct((1, 2), _f32),
)


def kernel(x, edge_index, W1, b1, W2, b2, W3, b3, S1, sb1, S2, sb2, S3, sb3,
           Wd1, bd1, Wd2, bd2):
    src = edge_index[0].astype(_i32)
    dst = edge_index[1].astype(_i32)
    pad = jnp.full((EP - EE,), NN, _i32)
    src_p = jnp.concatenate([src, pad])
    dst_p = jnp.concatenate([dst, pad])
    x_pad = jnp.concatenate([x, jnp.zeros((NP - NN, DD), _f32)], axis=0)

    src_t = src_p.reshape(NT, NCHUNK, CH)
    dst_t = dst_p.reshape(NT, NCHUNK, CH)
    degh = _k1()(src_t, dst_t)
    xs, ns_col, nd_col = _k2(degh.reshape(NC, 2, NP, 1), x_pad)
    def split3(a):
        p0 = a[:16 * NCH0 * CH].reshape(16, NCH0, CH)
        p0 = jnp.concatenate(
            [p0, jnp.full((16, NCHMAX - NCH0, CH), NN, _i32)], axis=1)
        p1 = a[16 * NCH0 * CH:].reshape(16, NCH1, CH)
        p1 = jnp.concatenate(
            [p1, jnp.full((16, NCHMAX - NCH1, CH), NN, _i32)], axis=1)
        return jnp.concatenate([p0, p1], axis=0)

    aggp = _k3()(split3(src_p), split3(dst_p), xs)
    s_col = _k4(aggp, nd_col, ns_col, W1, b1.reshape(1, HH),
                S1)
    shist = _k5()(src_t, dst_t, s_col.reshape(NP))
    topi, topv = _k6(shist.reshape(NC, NB, 128),
                     nd_col.reshape(NB, 128),
                     sb1.reshape(1, 1))
    idx32 = topi[0, :32]
    cp, agg32p, nd32, ns32 = _k7()(idx32, src_t, dst_t, aggp[0], aggp[1],
                                   nd_col.reshape(NP), ns_col.reshape(NP))
    out = _k8(cp.reshape(NC, 32, 32), agg32p, nd32.reshape(32, 1),
              ns32.reshape(32, 1), topv[0, :32].reshape(32, 1),
              W1, b1.reshape(1, HH), W2, b2.reshape(1, HH),
              W3, b3.reshape(1, HH), S2, sb2.reshape(1, 1),
              S3, sb3.reshape(1, 1), Wd1, bd1.reshape(1, 128),
              Wd2, bd2.reshape(1, 2))
    return out
